# baseline (speedup 1.0000x reference)
import jax
import jax.numpy as jnp
from jax import lax
from jax.experimental import pallas as pl
from jax.experimental.pallas import tpu as pltpu
from jax.experimental.pallas import tpu_sc as plsc

L_SEQ = 200
B_SEQ = 4096
D = 64
N_TOTAL = L_SEQ * B_SEQ
NC, NS = 2, 16
NW = NC * NS
IDX_MINOR = 128
N_ROWS = N_TOTAL // IDX_MINOR
ROWS_PER_W = N_ROWS // NW
K = 8
CHUNK = K * IDX_MINOR
N_CHUNKS = ROWS_PER_W // K


def _emb_kernel(idx_hbm, table_hbm, out_hbm, idx_v, rows_v, sem):
    wid = lax.axis_index("s") * NC + lax.axis_index("c")
    row_base = wid * ROWS_PER_W

    @pl.loop(0, N_CHUNKS)
    def _chunk(c):
        row0 = row_base + c * K
        pltpu.sync_copy(idx_hbm.at[pl.ds(row0, K), :], idx_v)
        descs = [
            pltpu.async_copy(
                table_hbm.at[idx_v.at[j]],
                rows_v.at[pl.ds(j * IDX_MINOR, IDX_MINOR), :],
                sem,
            )
            for j in range(K)
        ]
        for d in descs:
            d.wait()
        pltpu.sync_copy(rows_v, out_hbm.at[pl.ds(row0 * IDX_MINOR, CHUNK), :])


def kernel(sequence, W):
    idx = sequence.reshape(N_ROWS, IDX_MINOR)
    mesh = plsc.VectorSubcoreMesh(core_axis_name="c", subcore_axis_name="s")
    out = pl.kernel(
        _emb_kernel,
        out_type=jax.ShapeDtypeStruct((N_TOTAL, D), jnp.float32),
        mesh=mesh,
        scratch_types=[
            pltpu.VMEM((K, IDX_MINOR), jnp.int32),
            pltpu.VMEM((CHUNK, D), jnp.float32),
            pltpu.SemaphoreType.DMA,
        ],
        compiler_params=pltpu.CompilerParams(use_tc_tiling_on_sc=False),
    )(idx, W)
    return out.reshape(L_SEQ, B_SEQ, D)


# speedup vs baseline: 1.0143x; 1.0143x over previous
import jax
import jax.numpy as jnp
from jax import lax
from jax.experimental import pallas as pl
from jax.experimental.pallas import tpu as pltpu
from jax.experimental.pallas import tpu_sc as plsc

L_SEQ = 200
B_SEQ = 4096
D = 64
N_TOTAL = L_SEQ * B_SEQ
NC, NS = 2, 16
NW = NC * NS
IDX_MINOR = 128
N_ROWS = N_TOTAL // IDX_MINOR
ROWS_PER_W = N_ROWS // NW
K = 5
CHUNK = K * IDX_MINOR
N_CHUNKS = ROWS_PER_W // K
NBUF = 2
N_GROUPS = N_CHUNKS // NBUF


def _emb_kernel(idx_hbm, table_hbm, out_hbm, idx_all, r0, r1, g0, g1, o0, o1):
    rows = [r0, r1]
    gsem = [g0, g1]
    osem = [o0, o1]
    wid = lax.axis_index("s") * NC + lax.axis_index("c")
    row_base = wid * ROWS_PER_W
    out_base = row_base * IDX_MINOR

    pltpu.sync_copy(idx_hbm.at[pl.ds(row_base, ROWS_PER_W), :], idx_all)

    def fire(c, b):
        for j in range(K):
            pltpu.async_copy(
                table_hbm.at[idx_all.at[c * K + j]],
                rows[b].at[pl.ds(j * IDX_MINOR, IDX_MINOR), :],
                gsem[b],
            )

    def wait_gathers(b):
        pltpu.make_async_copy(out_hbm.at[pl.ds(0, CHUNK), :], rows[b], gsem[b]).wait()

    def start_out(c, b):
        pltpu.async_copy(rows[b], out_hbm.at[pl.ds(out_base + c * CHUNK, CHUNK), :], osem[b])

    def wait_out(b):
        pltpu.make_async_copy(rows[b], out_hbm.at[pl.ds(0, CHUNK), :], osem[b]).wait()

    for b in range(NBUF):
        fire(b, b)

    @pl.loop(0, N_GROUPS)
    def _group(g):
        for b in range(NBUF):
            c = g * NBUF + b
            wait_gathers(b)
            start_out(c, b)
        for b in range(NBUF):
            cn = g * NBUF + b + NBUF

            @pl.when(cn < N_CHUNKS)
            def _():
                wait_out(b)
                fire(cn, b)

    for b in range(NBUF):
        wait_out(b)


def kernel(sequence, W):
    idx = sequence.reshape(N_ROWS, IDX_MINOR)
    mesh = plsc.VectorSubcoreMesh(core_axis_name="c", subcore_axis_name="s")
    out = pl.kernel(
        _emb_kernel,
        out_type=jax.ShapeDtypeStruct((N_TOTAL, D), jnp.float32),
        mesh=mesh,
        scratch_types=[
            pltpu.VMEM((ROWS_PER_W, IDX_MINOR), jnp.int32),
            pltpu.VMEM((CHUNK, D), jnp.float32),
            pltpu.VMEM((CHUNK, D), jnp.float32),
            pltpu.SemaphoreType.DMA,
            pltpu.SemaphoreType.DMA,
            pltpu.SemaphoreType.DMA,
            pltpu.SemaphoreType.DMA,
        ],
        compiler_params=pltpu.CompilerParams(use_tc_tiling_on_sc=False),
    )(idx, W)
    return out.reshape(L_SEQ, B_SEQ, D)


# speedup vs baseline: 1.0152x; 1.0008x over previous
import jax
import jax.numpy as jnp
from jax import lax
from jax.experimental import pallas as pl
from jax.experimental.pallas import tpu as pltpu
from jax.experimental.pallas import tpu_sc as plsc

L_SEQ = 200
B_SEQ = 4096
D = 64
NC, NS = 2, 16
NW = NC * NS
IDX_MINOR = 128
K = 5
N_CHUNKS = L_SEQ // K
NBUF = 2
N_GROUPS = N_CHUNKS // NBUF


def _emb_kernel(idx_hbm, table_hbm, out_hbm, idx_all, r0, r1, g0, g1, o0, o1):
    rows = [r0, r1]
    gsem = [g0, g1]
    osem = [o0, o1]
    wid = lax.axis_index("s") * NC + lax.axis_index("c")
    col0 = wid * IDX_MINOR

    pltpu.sync_copy(idx_hbm.at[pl.ds(0, L_SEQ), pl.ds(col0, IDX_MINOR)], idx_all)

    def fire(c, b):
        for j in range(K):
            pltpu.async_copy(
                table_hbm.at[idx_all.at[c * K + j]],
                rows[b].at[j],
                gsem[b],
            )

    def wait_gathers(b):
        pltpu.make_async_copy(
            out_hbm.at[pl.ds(0, K), pl.ds(0, IDX_MINOR), :], rows[b], gsem[b]
        ).wait()

    def start_out(c, b):
        pltpu.async_copy(
            rows[b], out_hbm.at[pl.ds(c * K, K), pl.ds(col0, IDX_MINOR), :], osem[b]
        )

    def wait_out(b):
        pltpu.make_async_copy(
            rows[b], out_hbm.at[pl.ds(0, K), pl.ds(0, IDX_MINOR), :], osem[b]
        ).wait()

    for b in range(NBUF):
        fire(b, b)

    @pl.loop(0, N_GROUPS)
    def _group(g):
        for b in range(NBUF):
            c = g * NBUF + b
            wait_gathers(b)
            start_out(c, b)
        for b in range(NBUF):
            cn = g * NBUF + b + NBUF

            @pl.when(cn < N_CHUNKS)
            def _():
                wait_out(b)
                fire(cn, b)

    for b in range(NBUF):
        wait_out(b)


def kernel(sequence, W):
    mesh = plsc.VectorSubcoreMesh(core_axis_name="c", subcore_axis_name="s")
    return pl.kernel(
        _emb_kernel,
        out_type=jax.ShapeDtypeStruct((L_SEQ, B_SEQ, D), jnp.float32),
        mesh=mesh,
        scratch_types=[
            pltpu.VMEM((L_SEQ, IDX_MINOR), jnp.int32),
            pltpu.VMEM((K, IDX_MINOR, D), jnp.float32),
            pltpu.VMEM((K, IDX_MINOR, D), jnp.float32),
            pltpu.SemaphoreType.DMA,
            pltpu.SemaphoreType.DMA,
            pltpu.SemaphoreType.DMA,
            pltpu.SemaphoreType.DMA,
        ],
        compiler_params=pltpu.CompilerParams(use_tc_tiling_on_sc=False),
    )(sequence, W)


# speedup vs baseline: 1.2448x; 1.2262x over previous
import jax
import jax.numpy as jnp
from jax import lax
from jax.experimental import pallas as pl
from jax.experimental.pallas import tpu as pltpu
from jax.experimental.pallas import tpu_sc as plsc

L_SEQ = 200
B_SEQ = 4096
D = 64
DP = 128
VOCAB = 1000000
N_TOTAL = L_SEQ * B_SEQ
NC, NS = 2, 16
NW = NC * NS
IDX_MINOR = 128
K = 2
N_CHUNKS = L_SEQ // K
CHUNK = K * IDX_MINOR
NBUF = 2
N_GROUPS = N_CHUNKS // NBUF


def _emb_kernel(idx_hbm, table_hbm, out_hbm, idx_all, r0, r1, g0, g1, o0, o1):
    rows = [r0, r1]
    gsem = [g0, g1]
    osem = [o0, o1]
    wid = lax.axis_index("s") * NC + lax.axis_index("c")
    col0 = wid * IDX_MINOR

    pltpu.sync_copy(idx_hbm.at[pl.ds(0, L_SEQ), pl.ds(col0, IDX_MINOR)], idx_all)

    def fire(c, b):
        for j in range(K):
            pltpu.async_copy(
                table_hbm.at[idx_all.at[c * K + j]],
                rows[b].at[j],
                gsem[b],
            )

    def wait_gathers(b):
        pltpu.make_async_copy(
            out_hbm.at[pl.ds(0, CHUNK), :], rows[b].at[pl.ds(0, K), pl.ds(0, IDX_MINOR), :], gsem[b]
        ).wait()

    def start_out(c, b):
        for j in range(K):
            pltpu.async_copy(
                rows[b].at[j],
                out_hbm.at[pl.ds((c * K + j) * B_SEQ + col0, IDX_MINOR), :],
                osem[b],
            )

    def wait_out(b):
        pltpu.make_async_copy(
            rows[b].at[pl.ds(0, K), pl.ds(0, IDX_MINOR), :], out_hbm.at[pl.ds(0, CHUNK), :], osem[b]
        ).wait()

    for b in range(NBUF):
        fire(b, b)

    @pl.loop(0, N_GROUPS)
    def _group(g):
        for b in range(NBUF):
            c = g * NBUF + b
            wait_gathers(b)
            start_out(c, b)
        for b in range(NBUF):
            cn = g * NBUF + b + NBUF

            @pl.when(cn < N_CHUNKS)
            def _():
                wait_out(b)
                fire(cn, b)

    for b in range(NBUF):
        wait_out(b)


def kernel(sequence, W):
    W_pad = jnp.pad(W, ((0, 0), (0, DP - D)))
    mesh = plsc.VectorSubcoreMesh(core_axis_name="c", subcore_axis_name="s")
    out_pad = pl.kernel(
        _emb_kernel,
        out_type=jax.ShapeDtypeStruct((N_TOTAL, DP), jnp.float32),
        mesh=mesh,
        scratch_types=[
            pltpu.VMEM((L_SEQ, IDX_MINOR), jnp.int32),
            pltpu.VMEM((K, IDX_MINOR, DP), jnp.float32),
            pltpu.VMEM((K, IDX_MINOR, DP), jnp.float32),
            pltpu.SemaphoreType.DMA,
            pltpu.SemaphoreType.DMA,
            pltpu.SemaphoreType.DMA,
            pltpu.SemaphoreType.DMA,
        ],
        compiler_params=pltpu.CompilerParams(use_tc_tiling_on_sc=False),
    )(sequence, W_pad)
    return out_pad[:, :D].reshape(L_SEQ, B_SEQ, D)
